# Initial kernel scaffold; baseline (speedup 1.0000x reference)
import functools

import jax
import jax.numpy as jnp
from jax import lax
from jax.experimental import pallas as pl
from jax.experimental.pallas import tpu as pltpu
from jax.experimental.pallas import tpu_sc as plsc

_NC = 2
_NS = 16
_NW = _NC * _NS

_CHUNK = 400
_SUB = 80
_LANES = 16


def _gather_sum_call(E, N, D):
    e_per_w = E // _NW
    n_chunks = e_per_w // _CHUNK
    mesh = plsc.VectorSubcoreMesh(core_axis_name="c", subcore_axis_name="s")

    @functools.partial(
        pl.kernel,
        out_type=jax.ShapeDtypeStruct((E, D), jnp.float32),
        mesh=mesh,
        scratch_types=[
            pltpu.VMEM((_CHUNK,), jnp.int32),
            pltpu.VMEM((_CHUNK,), jnp.int32),
            pltpu.VMEM((_CHUNK, D), jnp.float32),
            pltpu.VMEM((_CHUNK, D), jnp.float32),
            pltpu.SemaphoreType.DMA,
        ],
    )
    def gather_sum(x_hbm, s_hbm, r_hbm, g_hbm, idx_s, idx_r, rows_s, rows_r, sem):
        wid = lax.axis_index("s") * _NC + lax.axis_index("c")
        w_base = wid * e_per_w

        def chunk_body(ci, carry):
            base = w_base + ci * _CHUNK
            pltpu.sync_copy(s_hbm.at[pl.ds(base, _CHUNK)], idx_s)
            pltpu.sync_copy(r_hbm.at[pl.ds(base, _CHUNK)], idx_r)
            cps = []
            for k in range(_CHUNK // _SUB):
                off = k * _SUB
                sl = pl.ds(off, _SUB)
                cps.append(
                    pltpu.async_copy(x_hbm.at[idx_s.at[sl]], rows_s.at[sl], sem)
                )
                cps.append(
                    pltpu.async_copy(x_hbm.at[idx_r.at[sl]], rows_r.at[sl], sem)
                )
            for cp in cps:
                cp.wait()

            def add_body(e, c2):
                for d in range(D // _LANES):
                    sl2 = pl.ds(d * _LANES, _LANES)
                    rows_s[e, sl2] = rows_s[e, sl2] + rows_r[e, sl2]
                return c2

            lax.fori_loop(0, _CHUNK, add_body, 0, unroll=2)
            pltpu.sync_copy(rows_s, g_hbm.at[pl.ds(base, _CHUNK)])
            return carry

        lax.fori_loop(0, n_chunks, chunk_body, 0)

    return gather_sum


def _combine_call(g, edge_attr, W, b2d, block_e):
    E, D = g.shape
    K = edge_attr.shape[1]

    def body(g_ref, ea_ref, w_ref, b_ref, o_ref):
        proj = (
            jnp.dot(ea_ref[...], w_ref[...], preferred_element_type=jnp.float32)
            + b_ref[...]
        )
        o_ref[...] = g_ref[...] * proj

    return pl.pallas_call(
        body,
        grid=(E // block_e,),
        in_specs=[
            pl.BlockSpec((block_e, D), lambda i: (i, 0)),
            pl.BlockSpec((block_e, K), lambda i: (i, 0)),
            pl.BlockSpec((K, D), lambda i: (0, 0)),
            pl.BlockSpec((1, D), lambda i: (0, 0)),
        ],
        out_specs=pl.BlockSpec((block_e, D), lambda i: (i, 0)),
        out_shape=jax.ShapeDtypeStruct((E, D), jnp.float32),
    )(g, edge_attr, W, b2d)


def kernel(senders, receivers, edge_attr, x, W, b):
    E = senders.shape[0]
    N, D = x.shape
    senders = senders.astype(jnp.int32)
    receivers = receivers.astype(jnp.int32)
    g = _gather_sum_call(E, N, D)(x, senders, receivers)
    return _combine_call(g, edge_attr, W, b.reshape(1, D), block_e=512)


# speedup vs baseline: 1.5399x; 1.5399x over previous
import functools

import jax
import jax.numpy as jnp
from jax import lax
from jax.experimental import pallas as pl
from jax.experimental.pallas import tpu as pltpu
from jax.experimental.pallas import tpu_sc as plsc

_NC = 2
_NS = 16
_NW = _NC * _NS

_CHUNK = 400
_SUB = 80
_LANES = 16


def _gather_sum_call(E, N, D):
    e_per_w = E // _NW
    n_chunks = e_per_w // _CHUNK
    mesh = plsc.VectorSubcoreMesh(
        core_axis_name="c", subcore_axis_name="s", num_cores=_NC, num_subcores=_NS
    )

    @functools.partial(
        pl.kernel,
        out_type=jax.ShapeDtypeStruct((E, D), jnp.float32),
        mesh=mesh,
        scratch_types=[
            pltpu.VMEM((_CHUNK,), jnp.int32),
            pltpu.VMEM((_CHUNK,), jnp.int32),
            pltpu.VMEM((_CHUNK, D), jnp.float32),
            pltpu.VMEM((_CHUNK, D), jnp.float32),
            pltpu.SemaphoreType.DMA,
        ],
    )
    def gather_sum(x_hbm, s_hbm, r_hbm, g_hbm, idx_s, idx_r, rows_s, rows_r, sem):
        wid = lax.axis_index("s") * _NC + lax.axis_index("c")
        w_base = wid * e_per_w

        def chunk_body(ci, carry):
            base = w_base + ci * _CHUNK
            pltpu.sync_copy(s_hbm.at[pl.ds(base, _CHUNK)], idx_s)
            pltpu.sync_copy(r_hbm.at[pl.ds(base, _CHUNK)], idx_r)
            cps = []
            for k in range(_CHUNK // _SUB):
                off = k * _SUB
                sl = pl.ds(off, _SUB)
                cps.append(
                    pltpu.async_copy(x_hbm.at[idx_s.at[sl]], rows_s.at[sl], sem)
                )
                cps.append(
                    pltpu.async_copy(x_hbm.at[idx_r.at[sl]], rows_r.at[sl], sem)
                )
            for cp in cps:
                cp.wait()

            def add_body(e, c2):
                for d in range(D // _LANES):
                    sl2 = pl.ds(d * _LANES, _LANES)
                    rows_s[e, sl2] = rows_s[e, sl2] + rows_r[e, sl2]
                return c2

            lax.fori_loop(0, _CHUNK, add_body, 0, unroll=2)
            pltpu.sync_copy(rows_s, g_hbm.at[pl.ds(base, _CHUNK)])
            return carry

        lax.fori_loop(0, n_chunks, chunk_body, 0)

    return gather_sum


def _combine_call(g, edge_attr, W, b2d, block_e):
    E, D = g.shape
    K = edge_attr.shape[1]

    def body(g_ref, ea_ref, w_ref, b_ref, o_ref):
        proj = (
            jnp.dot(ea_ref[...], w_ref[...], preferred_element_type=jnp.float32)
            + b_ref[...]
        )
        o_ref[...] = g_ref[...] * proj

    return pl.pallas_call(
        body,
        grid=(E // block_e,),
        in_specs=[
            pl.BlockSpec((block_e, D), lambda i: (i, 0)),
            pl.BlockSpec((block_e, K), lambda i: (i, 0)),
            pl.BlockSpec((K, D), lambda i: (0, 0)),
            pl.BlockSpec((1, D), lambda i: (0, 0)),
        ],
        out_specs=pl.BlockSpec((block_e, D), lambda i: (i, 0)),
        out_shape=jax.ShapeDtypeStruct((E, D), jnp.float32),
    )(g, edge_attr, W, b2d)


def kernel(senders, receivers, edge_attr, x, W, b):
    E = senders.shape[0]
    N, D = x.shape
    senders = senders.astype(jnp.int32)
    receivers = receivers.astype(jnp.int32)
    g = _gather_sum_call(E, N, D)(x, senders, receivers)
    return _combine_call(g, edge_attr, W, b.reshape(1, D), block_e=512)


# speedup vs baseline: 3.1930x; 2.0736x over previous
import functools

import jax
import jax.numpy as jnp
from jax import lax
from jax.experimental import pallas as pl
from jax.experimental.pallas import tpu as pltpu
from jax.experimental.pallas import tpu_sc as plsc

_NC = 2
_NS = 16
_NW = _NC * _NS

_CHUNK = 200
_SUB = 40
_LANES = 16


def _gather_sum_call(E, N, D):
    epw = E // _NW
    nch = epw // _CHUNK
    assert nch % 2 == 0
    mesh = plsc.VectorSubcoreMesh(
        core_axis_name="c", subcore_axis_name="s", num_cores=_NC, num_subcores=_NS
    )

    @functools.partial(
        pl.kernel,
        out_type=jax.ShapeDtypeStruct((E, D), jnp.float32),
        mesh=mesh,
        scratch_types=[
            pltpu.VMEM((epw,), jnp.int32),
            pltpu.VMEM((epw,), jnp.int32),
            pltpu.VMEM((2, _CHUNK, D), jnp.float32),
            pltpu.VMEM((2, _CHUNK, D), jnp.float32),
            pltpu.SemaphoreType.DMA,
            pltpu.SemaphoreType.DMA,
            pltpu.SemaphoreType.DMA,
            pltpu.SemaphoreType.DMA,
        ],
    )
    def gather_sum(
        x_hbm, s_hbm, r_hbm, g_hbm, idxs, idxr, rows_s, rows_r, g0, g1, w0, w1
    ):
        wid = lax.axis_index("s") * _NC + lax.axis_index("c")
        w_base = wid * epw
        gsem = (g0, g1)
        wsem = (w0, w1)

        pltpu.sync_copy(s_hbm.at[pl.ds(w_base, epw)], idxs)
        pltpu.sync_copy(r_hbm.at[pl.ds(w_base, epw)], idxr)

        def start_gathers(ci, p):
            off = ci * _CHUNK
            for k in range(_CHUNK // _SUB):
                sl = pl.ds(off + k * _SUB, _SUB)
                dsl = pl.ds(k * _SUB, _SUB)
                pltpu.async_copy(x_hbm.at[idxs.at[sl]], rows_s.at[p, dsl], gsem[p])
                pltpu.async_copy(x_hbm.at[idxr.at[sl]], rows_r.at[p, dsl], gsem[p])

        def wait_gathers(p):
            pltpu.make_async_copy(
                x_hbm.at[idxs.at[pl.ds(0, _CHUNK)]], rows_s.at[p], gsem[p]
            ).wait()
            pltpu.make_async_copy(
                x_hbm.at[idxr.at[pl.ds(0, _CHUNK)]], rows_r.at[p], gsem[p]
            ).wait()

        def start_writeback(ci, p):
            pltpu.async_copy(
                rows_s.at[p], g_hbm.at[pl.ds(w_base + ci * _CHUNK, _CHUNK)], wsem[p]
            )

        def wait_writeback(p):
            pltpu.make_async_copy(
                rows_s.at[p], g_hbm.at[pl.ds(w_base, _CHUNK)], wsem[p]
            ).wait()

        def add_rows(p):
            def add_body(e, c2):
                for d in range(D // _LANES):
                    sl2 = pl.ds(d * _LANES, _LANES)
                    rows_s[p, e, sl2] = rows_s[p, e, sl2] + rows_r[p, e, sl2]
                return c2

            lax.fori_loop(0, _CHUNK, add_body, 0, unroll=4)

        start_gathers(0, 0)

        def super_body(j, carry):
            @pl.when(j > 0)
            def _():
                wait_writeback(1)

            start_gathers(2 * j + 1, 1)
            wait_gathers(0)
            add_rows(0)
            start_writeback(2 * j, 0)

            wait_writeback(0)

            @pl.when(2 * j + 2 < nch)
            def _():
                start_gathers(2 * j + 2, 0)

            wait_gathers(1)
            add_rows(1)
            start_writeback(2 * j + 1, 1)
            return carry

        lax.fori_loop(0, nch // 2, super_body, 0)
        wait_writeback(1)

    return gather_sum


def _combine_call(g, edge_attr, W, b2d, block_e):
    E, D = g.shape
    K = edge_attr.shape[1]

    def body(g_ref, ea_ref, w_ref, b_ref, o_ref):
        proj = (
            jnp.dot(ea_ref[...], w_ref[...], preferred_element_type=jnp.float32)
            + b_ref[...]
        )
        o_ref[...] = g_ref[...] * proj

    return pl.pallas_call(
        body,
        grid=(E // block_e,),
        in_specs=[
            pl.BlockSpec((block_e, D), lambda i: (i, 0)),
            pl.BlockSpec((block_e, K), lambda i: (i, 0)),
            pl.BlockSpec((K, D), lambda i: (0, 0)),
            pl.BlockSpec((1, D), lambda i: (0, 0)),
        ],
        out_specs=pl.BlockSpec((block_e, D), lambda i: (i, 0)),
        out_shape=jax.ShapeDtypeStruct((E, D), jnp.float32),
    )(g, edge_attr, W, b2d)


def kernel(senders, receivers, edge_attr, x, W, b):
    E = senders.shape[0]
    N, D = x.shape
    senders = senders.astype(jnp.int32)
    receivers = receivers.astype(jnp.int32)
    g = _gather_sum_call(E, N, D)(x, senders, receivers)
    return _combine_call(g, edge_attr, W, b.reshape(1, D), block_e=2560)


# speedup vs baseline: 3.4350x; 1.0758x over previous
import functools

import jax
import jax.numpy as jnp
from jax import lax
from jax.experimental import pallas as pl
from jax.experimental.pallas import tpu as pltpu
from jax.experimental.pallas import tpu_sc as plsc

_NC = 2
_NS = 16
_NW = _NC * _NS

_CHUNK = 128
_RING = 3
_LANES = 16


def _gather_sum_call(E, N, D):
    epw = E // _NW
    n_full = epw // _CHUNK
    tail = epw - n_full * _CHUNK
    assert n_full % _RING == 0 and tail < _CHUNK
    mesh = plsc.VectorSubcoreMesh(
        core_axis_name="c", subcore_axis_name="s", num_cores=_NC, num_subcores=_NS
    )

    @functools.partial(
        pl.kernel,
        out_type=jax.ShapeDtypeStruct((E, D), jnp.float32),
        mesh=mesh,
        scratch_types=[
            pltpu.VMEM((epw,), jnp.int32),
            pltpu.VMEM((epw,), jnp.int32),
            pltpu.VMEM((_RING, _CHUNK, D), jnp.float32),
            pltpu.VMEM((_RING, _CHUNK, D), jnp.float32),
            pltpu.SemaphoreType.DMA,
            pltpu.SemaphoreType.DMA,
            pltpu.SemaphoreType.DMA,
            pltpu.SemaphoreType.DMA,
            pltpu.SemaphoreType.DMA,
            pltpu.SemaphoreType.DMA,
        ],
    )
    def gather_sum(
        x_hbm, s_hbm, r_hbm, g_hbm, idxs, idxr, rows_s, rows_r,
        g0, g1, g2, w0, w1, w2,
    ):
        wid = lax.axis_index("s") * _NC + lax.axis_index("c")
        w_base = wid * epw
        gsem = (g0, g1, g2)
        wsem = (w0, w1, w2)

        pltpu.sync_copy(s_hbm.at[pl.ds(w_base, epw)], idxs)
        pltpu.sync_copy(r_hbm.at[pl.ds(w_base, epw)], idxr)

        def start_gathers(ci, p, n=_CHUNK):
            sl = pl.ds(ci * _CHUNK, n)
            pltpu.async_copy(x_hbm.at[idxs.at[sl]], rows_s.at[p, pl.ds(0, n)], gsem[p])
            pltpu.async_copy(x_hbm.at[idxr.at[sl]], rows_r.at[p, pl.ds(0, n)], gsem[p])

        def wait_gathers(p, n=_CHUNK):
            sl = pl.ds(0, n)
            pltpu.make_async_copy(
                x_hbm.at[idxs.at[sl]], rows_s.at[p, sl], gsem[p]
            ).wait()
            pltpu.make_async_copy(
                x_hbm.at[idxr.at[sl]], rows_r.at[p, sl], gsem[p]
            ).wait()

        def start_writeback(ci, p, n=_CHUNK):
            pltpu.async_copy(
                rows_s.at[p, pl.ds(0, n)],
                g_hbm.at[pl.ds(w_base + ci * _CHUNK, n)],
                wsem[p],
            )

        def wait_writeback(p, n=_CHUNK):
            pltpu.make_async_copy(
                rows_s.at[p, pl.ds(0, n)], g_hbm.at[pl.ds(w_base, n)], wsem[p]
            ).wait()

        def add_rows(p, n=_CHUNK):
            def add_body(e, c2):
                for d in range(D // _LANES):
                    sl2 = pl.ds(d * _LANES, _LANES)
                    rows_s[p, e, sl2] = rows_s[p, e, sl2] + rows_r[p, e, sl2]
                return c2

            lax.fori_loop(0, n, add_body, 0, unroll=4)

        start_gathers(0, 0)
        start_gathers(1, 1)

        def super_body(j, carry):
            for b in range(_RING):
                i = _RING * j + b
                pn = (b + 2) % _RING

                @pl.when(i >= 1)
                def _():
                    wait_writeback(pn)

                @pl.when(i + 2 < n_full)
                def _():
                    start_gathers(i + 2, pn)

                wait_gathers(b)
                add_rows(b)
                start_writeback(i, b)
            return carry

        lax.fori_loop(0, n_full // _RING, super_body, 0)
        if tail:
            start_gathers(n_full, 0, tail)
            wait_gathers(0, tail)
            add_rows(0, tail)
            start_writeback(n_full, 0, tail)
            wait_writeback(0, tail)
        wait_writeback((n_full - 1) % _RING)

    return gather_sum


def _combine_call(g, edge_attr, W, b2d, block_e):
    E, D = g.shape
    K = edge_attr.shape[1]

    def body(g_ref, ea_ref, w_ref, b_ref, o_ref):
        proj = (
            jnp.dot(ea_ref[...], w_ref[...], preferred_element_type=jnp.float32)
            + b_ref[...]
        )
        o_ref[...] = g_ref[...] * proj

    return pl.pallas_call(
        body,
        grid=(E // block_e,),
        in_specs=[
            pl.BlockSpec((block_e, D), lambda i: (i, 0)),
            pl.BlockSpec((block_e, K), lambda i: (i, 0)),
            pl.BlockSpec((K, D), lambda i: (0, 0)),
            pl.BlockSpec((1, D), lambda i: (0, 0)),
        ],
        out_specs=pl.BlockSpec((block_e, D), lambda i: (i, 0)),
        out_shape=jax.ShapeDtypeStruct((E, D), jnp.float32),
    )(g, edge_attr, W, b2d)


def kernel(senders, receivers, edge_attr, x, W, b):
    E = senders.shape[0]
    N, D = x.shape
    senders = senders.astype(jnp.int32)
    receivers = receivers.astype(jnp.int32)
    g = _gather_sum_call(E, N, D)(x, senders, receivers)
    return _combine_call(g, edge_attr, W, b.reshape(1, D), block_e=2560)


# speedup vs baseline: 3.6434x; 1.0607x over previous
import functools

import jax
import jax.numpy as jnp
from jax import lax
from jax.experimental import pallas as pl
from jax.experimental.pallas import tpu as pltpu
from jax.experimental.pallas import tpu_sc as plsc

_NC = 2
_NS = 16
_NW = _NC * _NS

_CHUNK = 128
_RING = 3
_LANES = 16


def _gather_sum_call(E, N, D):
    epw = E // _NW
    n_full = epw // _CHUNK
    tail = epw - n_full * _CHUNK
    assert n_full % _RING == 0 and tail < _CHUNK
    mesh = plsc.VectorSubcoreMesh(
        core_axis_name="c", subcore_axis_name="s", num_cores=_NC, num_subcores=_NS
    )

    @functools.partial(
        pl.kernel,
        out_type=jax.ShapeDtypeStruct((E, D), jnp.float32),
        mesh=mesh,
        scratch_types=[
            pltpu.VMEM((epw,), jnp.int32),
            pltpu.VMEM((epw,), jnp.int32),
            pltpu.VMEM((_RING, _CHUNK, D), jnp.float32),
            pltpu.VMEM((_RING, _CHUNK, D), jnp.float32),
            pltpu.SemaphoreType.DMA,
            pltpu.SemaphoreType.DMA,
            pltpu.SemaphoreType.DMA,
            pltpu.SemaphoreType.DMA,
            pltpu.SemaphoreType.DMA,
            pltpu.SemaphoreType.DMA,
        ],
    )
    def gather_sum(
        x_hbm, s_hbm, r_hbm, g_hbm, idxs, idxr, rows_s, rows_r,
        g0, g1, g2, w0, w1, w2,
    ):
        wid = lax.axis_index("s") * _NC + lax.axis_index("c")
        w_base = wid * epw
        gsem = (g0, g1, g2)
        wsem = (w0, w1, w2)

        pltpu.sync_copy(s_hbm.at[pl.ds(w_base, epw)], idxs)
        pltpu.sync_copy(r_hbm.at[pl.ds(w_base, epw)], idxr)

        def start_gathers(ci, p, n=_CHUNK):
            sl = pl.ds(ci * _CHUNK, n)
            pltpu.async_copy(x_hbm.at[idxs.at[sl]], rows_s.at[p, pl.ds(0, n)], gsem[p])
            pltpu.async_copy(x_hbm.at[idxr.at[sl]], rows_r.at[p, pl.ds(0, n)], gsem[p])

        def wait_gathers(p, n=_CHUNK):
            sl = pl.ds(0, n)
            pltpu.make_async_copy(
                x_hbm.at[idxs.at[sl]], rows_s.at[p, sl], gsem[p]
            ).wait()
            pltpu.make_async_copy(
                x_hbm.at[idxr.at[sl]], rows_r.at[p, sl], gsem[p]
            ).wait()

        def start_writeback(ci, p, n=_CHUNK):
            pltpu.async_copy(
                rows_s.at[p, pl.ds(0, n)],
                g_hbm.at[pl.ds(w_base + ci * _CHUNK, n)],
                wsem[p],
            )

        def wait_writeback(p, n=_CHUNK):
            pltpu.make_async_copy(
                rows_s.at[p, pl.ds(0, n)], g_hbm.at[pl.ds(w_base, n)], wsem[p]
            ).wait()

        def add_rows(p, n=_CHUNK):
            def add_body(e, c2):
                for d in range(D // _LANES):
                    sl2 = pl.ds(d * _LANES, _LANES)
                    rows_s[p, e, sl2] = rows_s[p, e, sl2] + rows_r[p, e, sl2]
                return c2

            lax.fori_loop(0, n, add_body, 0, unroll=4)

        start_gathers(0, 0)
        start_gathers(1, 1)

        def super_body(j, carry):
            for b in range(_RING):
                i = _RING * j + b
                pn = (b + 2) % _RING

                @pl.when(i >= 1)
                def _():
                    wait_writeback(pn)

                @pl.when(i + 2 < n_full)
                def _():
                    start_gathers(i + 2, pn)

                wait_gathers(b)
                add_rows(b)
                start_writeback(i, b)
            return carry

        lax.fori_loop(0, n_full // _RING, super_body, 0)
        if tail:
            start_gathers(n_full, 0, tail)
            wait_gathers(0, tail)
            add_rows(0, tail)
            start_writeback(n_full, 0, tail)
            wait_writeback(0, tail)
        wait_writeback((n_full - 1) % _RING)

    return gather_sum


def _combine_slab(g_slab, ea_slab, W, b2d, t_prev, E, off_blocks, block_e):
    Es, D = g_slab.shape
    K = ea_slab.shape[1]

    if t_prev is None:
        def body(g_ref, ea_ref, w_ref, b_ref, o_ref):
            proj = (
                jnp.dot(ea_ref[...], w_ref[...], preferred_element_type=jnp.float32)
                + b_ref[...]
            )
            o_ref[...] = g_ref[...] * proj
        extra_in, extra_specs, aliases = (), (), {}
    else:
        def body(g_ref, ea_ref, w_ref, b_ref, t_ref, o_ref):
            del t_ref
            proj = (
                jnp.dot(ea_ref[...], w_ref[...], preferred_element_type=jnp.float32)
                + b_ref[...]
            )
            o_ref[...] = g_ref[...] * proj
        extra_in = (t_prev,)
        extra_specs = (pl.BlockSpec(memory_space=pl.ANY),)
        aliases = {4: 0}

    return pl.pallas_call(
        body,
        grid=(Es // block_e,),
        in_specs=[
            pl.BlockSpec((block_e, D), lambda i: (i, 0)),
            pl.BlockSpec((block_e, K), lambda i: (i, 0)),
            pl.BlockSpec((K, D), lambda i: (0, 0)),
            pl.BlockSpec((1, D), lambda i: (0, 0)),
            *extra_specs,
        ],
        out_specs=pl.BlockSpec((block_e, D), lambda i: (i + off_blocks, 0)),
        out_shape=jax.ShapeDtypeStruct((E, D), jnp.float32),
        input_output_aliases=aliases,
    )(g_slab, ea_slab, W, b2d, *extra_in)


_N_SLABS = 2
_BLOCK_E = 3200


def kernel(senders, receivers, edge_attr, x, W, b):
    E = senders.shape[0]
    N, D = x.shape
    senders = senders.astype(jnp.int32)
    receivers = receivers.astype(jnp.int32)
    b2d = b.reshape(1, D)

    Es = E // _N_SLABS
    sc = _gather_sum_call(Es, N, D)
    gs = [
        sc(x, senders[s * Es:(s + 1) * Es], receivers[s * Es:(s + 1) * Es])
        for s in range(_N_SLABS)
    ]
    blocks_per_slab = Es // _BLOCK_E
    t = None
    for s in range(_N_SLABS):
        t = _combine_slab(
            gs[s], edge_attr[s * Es:(s + 1) * Es], W, b2d, t,
            E, s * blocks_per_slab, _BLOCK_E,
        )
    return t


# speedup vs baseline: 3.7694x; 1.0346x over previous
import functools

import jax
import jax.numpy as jnp
from jax import lax
from jax.experimental import pallas as pl
from jax.experimental.pallas import tpu as pltpu
from jax.experimental.pallas import tpu_sc as plsc

_NC = 2
_NS = 16
_NW = _NC * _NS

_CHUNK = 128
_RING = 3
_LANES = 16


def _gather_sum_call(E, N, D):
    epw = E // _NW
    n_full = epw // _CHUNK
    tail = epw - n_full * _CHUNK
    assert n_full % _RING == 0 and tail < _CHUNK
    mesh = plsc.VectorSubcoreMesh(
        core_axis_name="c", subcore_axis_name="s", num_cores=_NC, num_subcores=_NS
    )

    @functools.partial(
        pl.kernel,
        out_type=jax.ShapeDtypeStruct((E, D), jnp.float32),
        mesh=mesh,
        scratch_types=[
            pltpu.VMEM((epw,), jnp.int32),
            pltpu.VMEM((epw,), jnp.int32),
            pltpu.VMEM((_RING, _CHUNK, D), jnp.float32),
            pltpu.SemaphoreType.DMA,
            pltpu.SemaphoreType.DMA,
            pltpu.SemaphoreType.DMA,
            pltpu.SemaphoreType.DMA,
            pltpu.SemaphoreType.DMA,
            pltpu.SemaphoreType.DMA,
            pltpu.SemaphoreType.DMA,
            pltpu.SemaphoreType.DMA,
            pltpu.SemaphoreType.DMA,
        ],
    )
    def gather_sum(
        x_hbm, s_hbm, r_hbm, g_hbm, idxs, idxr, rows,
        s0, s1, s2, a0, a1, a2, w0, w1, w2,
    ):
        wid = lax.axis_index("s") * _NC + lax.axis_index("c")
        w_base = wid * epw
        ssem = (s0, s1, s2)
        asem = (a0, a1, a2)
        wsem = (w0, w1, w2)

        pltpu.sync_copy(s_hbm.at[pl.ds(w_base, epw)], idxs)
        pltpu.sync_copy(r_hbm.at[pl.ds(w_base, epw)], idxr)

        def gs(ci, p, n=_CHUNK):
            pltpu.async_copy(
                x_hbm.at[idxs.at[pl.ds(ci * _CHUNK, n)]],
                rows.at[p, pl.ds(0, n)],
                ssem[p],
            )

        def wait_gs(p, n=_CHUNK):
            pltpu.make_async_copy(
                x_hbm.at[idxs.at[pl.ds(0, n)]], rows.at[p, pl.ds(0, n)], ssem[p]
            ).wait()

        def ga(ci, p, n=_CHUNK):
            pltpu.async_copy(
                x_hbm.at[idxr.at[pl.ds(ci * _CHUNK, n)]],
                rows.at[p, pl.ds(0, n)],
                asem[p],
                add=True,
            )

        def wait_ga(p, n=_CHUNK):
            pltpu.make_async_copy(
                x_hbm.at[idxr.at[pl.ds(0, n)]], rows.at[p, pl.ds(0, n)], asem[p]
            ).wait()

        def wb(ci, p, n=_CHUNK):
            pltpu.async_copy(
                rows.at[p, pl.ds(0, n)],
                g_hbm.at[pl.ds(w_base + ci * _CHUNK, n)],
                wsem[p],
            )

        def wait_wb(p, n=_CHUNK):
            pltpu.make_async_copy(
                rows.at[p, pl.ds(0, n)], g_hbm.at[pl.ds(w_base, n)], wsem[p]
            ).wait()

        gs(0, 0)
        gs(1, 1)
        wait_gs(0)
        ga(0, 0)

        def super_body(j, carry):
            for b in range(_RING):
                i = _RING * j + b
                p1 = (b + 1) % _RING
                p2 = (b + 2) % _RING

                @pl.when(i + 1 < n_full)
                def _():
                    wait_gs(p1)
                    ga(i + 1, p1)

                @pl.when((i >= 1) & (i + 2 < n_full))
                def _():
                    wait_wb(p2)

                @pl.when(i + 2 < n_full)
                def _():
                    gs(i + 2, p2)

                wait_ga(b)
                wb(i, b)
            return carry

        lax.fori_loop(0, n_full // _RING, super_body, 0)
        wait_wb((n_full - 3) % _RING)
        wait_wb((n_full - 2) % _RING)
        if tail:
            gs(n_full, 0, tail)
            wait_gs(0, tail)
            ga(n_full, 0, tail)
            wait_ga(0, tail)
            wb(n_full, 0, tail)
            wait_wb(0, tail)
        wait_wb((n_full - 1) % _RING)

    return gather_sum


def _combine_slab(g_slab, ea_slab, W, b2d, t_prev, E, off_blocks, block_e):
    Es, D = g_slab.shape
    K = ea_slab.shape[1]

    if t_prev is None:
        def body(g_ref, ea_ref, w_ref, b_ref, o_ref):
            proj = (
                jnp.dot(ea_ref[...], w_ref[...], preferred_element_type=jnp.float32)
                + b_ref[...]
            )
            o_ref[...] = g_ref[...] * proj
        extra_in, extra_specs, aliases = (), (), {}
    else:
        def body(g_ref, ea_ref, w_ref, b_ref, t_ref, o_ref):
            del t_ref
            proj = (
                jnp.dot(ea_ref[...], w_ref[...], preferred_element_type=jnp.float32)
                + b_ref[...]
            )
            o_ref[...] = g_ref[...] * proj
        extra_in = (t_prev,)
        extra_specs = (pl.BlockSpec(memory_space=pl.ANY),)
        aliases = {4: 0}

    return pl.pallas_call(
        body,
        grid=(Es // block_e,),
        in_specs=[
            pl.BlockSpec((block_e, D), lambda i: (i, 0)),
            pl.BlockSpec((block_e, K), lambda i: (i, 0)),
            pl.BlockSpec((K, D), lambda i: (0, 0)),
            pl.BlockSpec((1, D), lambda i: (0, 0)),
            *extra_specs,
        ],
        out_specs=pl.BlockSpec((block_e, D), lambda i: (i + off_blocks, 0)),
        out_shape=jax.ShapeDtypeStruct((E, D), jnp.float32),
        input_output_aliases=aliases,
    )(g_slab, ea_slab, W, b2d, *extra_in)


_N_SLABS = 2
_BLOCK_E = 3200


def kernel(senders, receivers, edge_attr, x, W, b):
    E = senders.shape[0]
    N, D = x.shape
    senders = senders.astype(jnp.int32)
    receivers = receivers.astype(jnp.int32)
    b2d = b.reshape(1, D)

    Es = E // _N_SLABS
    sc = _gather_sum_call(Es, N, D)
    gs = [
        sc(x, senders[s * Es:(s + 1) * Es], receivers[s * Es:(s + 1) * Es])
        for s in range(_N_SLABS)
    ]
    blocks_per_slab = Es // _BLOCK_E
    t = None
    for s in range(_N_SLABS):
        t = _combine_slab(
            gs[s], edge_attr[s * Es:(s + 1) * Es], W, b2d, t,
            E, s * blocks_per_slab, _BLOCK_E,
        )
    return t


# speedup vs baseline: 3.8251x; 1.0148x over previous
import functools

import jax
import jax.numpy as jnp
from jax import lax
from jax.experimental import pallas as pl
from jax.experimental.pallas import tpu as pltpu
from jax.experimental.pallas import tpu_sc as plsc

_NC = 2
_NS = 16
_NW = _NC * _NS

_CHUNK = 128
_RING = 3
_LANES = 16


def _gather_sum_call(E, N, D):
    epw = E // _NW
    n_full = epw // _CHUNK
    tail = epw - n_full * _CHUNK
    assert n_full % _RING == 0 and tail < _CHUNK
    mesh = plsc.VectorSubcoreMesh(
        core_axis_name="c", subcore_axis_name="s", num_cores=_NC, num_subcores=_NS
    )

    @functools.partial(
        pl.kernel,
        out_type=jax.ShapeDtypeStruct((E, D), jnp.float32),
        mesh=mesh,
        scratch_types=[
            pltpu.VMEM((epw,), jnp.int32),
            pltpu.VMEM((epw,), jnp.int32),
            pltpu.VMEM((_RING, _CHUNK, D), jnp.float32),
            pltpu.SemaphoreType.DMA,
            pltpu.SemaphoreType.DMA,
            pltpu.SemaphoreType.DMA,
            pltpu.SemaphoreType.DMA,
            pltpu.SemaphoreType.DMA,
            pltpu.SemaphoreType.DMA,
            pltpu.SemaphoreType.DMA,
            pltpu.SemaphoreType.DMA,
            pltpu.SemaphoreType.DMA,
        ],
    )
    def gather_sum(
        x_hbm, s_hbm, r_hbm, g_hbm, idxs, idxr, rows,
        s0, s1, s2, a0, a1, a2, w0, w1, w2,
    ):
        wid = lax.axis_index("s") * _NC + lax.axis_index("c")
        w_base = wid * epw
        ssem = (s0, s1, s2)
        asem = (a0, a1, a2)
        wsem = (w0, w1, w2)

        pltpu.sync_copy(s_hbm.at[pl.ds(w_base, epw)], idxs)
        pltpu.sync_copy(r_hbm.at[pl.ds(w_base, epw)], idxr)

        def gs(ci, p, n=_CHUNK):
            pltpu.async_copy(
                x_hbm.at[idxs.at[pl.ds(ci * _CHUNK, n)]],
                rows.at[p, pl.ds(0, n)],
                ssem[p],
            )

        def wait_gs(p, n=_CHUNK):
            pltpu.make_async_copy(
                x_hbm.at[idxs.at[pl.ds(0, n)]], rows.at[p, pl.ds(0, n)], ssem[p]
            ).wait()

        def ga(ci, p, n=_CHUNK):
            pltpu.async_copy(
                x_hbm.at[idxr.at[pl.ds(ci * _CHUNK, n)]],
                rows.at[p, pl.ds(0, n)],
                asem[p],
                add=True,
            )

        def wait_ga(p, n=_CHUNK):
            pltpu.make_async_copy(
                x_hbm.at[idxr.at[pl.ds(0, n)]], rows.at[p, pl.ds(0, n)], asem[p]
            ).wait()

        def wb(ci, p, n=_CHUNK):
            pltpu.async_copy(
                rows.at[p, pl.ds(0, n)],
                g_hbm.at[pl.ds(w_base + ci * _CHUNK, n)],
                wsem[p],
            )

        def wait_wb(p, n=_CHUNK):
            pltpu.make_async_copy(
                rows.at[p, pl.ds(0, n)], g_hbm.at[pl.ds(w_base, n)], wsem[p]
            ).wait()

        gs(0, 0)
        gs(1, 1)
        wait_gs(0)
        ga(0, 0)

        def super_body(j, carry):
            for b in range(_RING):
                i = _RING * j + b
                p1 = (b + 1) % _RING
                p2 = (b + 2) % _RING

                @pl.when(i + 1 < n_full)
                def _():
                    wait_gs(p1)
                    ga(i + 1, p1)

                @pl.when((i >= 1) & (i + 2 < n_full))
                def _():
                    wait_wb(p2)

                @pl.when(i + 2 < n_full)
                def _():
                    gs(i + 2, p2)

                wait_ga(b)
                wb(i, b)
            return carry

        lax.fori_loop(0, n_full // _RING, super_body, 0)
        wait_wb((n_full - 3) % _RING)
        wait_wb((n_full - 2) % _RING)
        if tail:
            gs(n_full, 0, tail)
            wait_gs(0, tail)
            ga(n_full, 0, tail)
            wait_ga(0, tail)
            wb(n_full, 0, tail)
            wait_wb(0, tail)
        wait_wb((n_full - 1) % _RING)

    return gather_sum


def _combine_slab(g_slab, ea_slab, W, b2d, t_prev, E, off_blocks, block_e):
    Es, D = g_slab.shape
    K = ea_slab.shape[1]

    if t_prev is None:
        def body(g_ref, ea_ref, w_ref, b_ref, o_ref):
            proj = (
                jnp.dot(ea_ref[...], w_ref[...], preferred_element_type=jnp.float32)
                + b_ref[...]
            )
            o_ref[...] = g_ref[...] * proj
        extra_in, extra_specs, aliases = (), (), {}
    else:
        def body(g_ref, ea_ref, w_ref, b_ref, t_ref, o_ref):
            del t_ref
            proj = (
                jnp.dot(ea_ref[...], w_ref[...], preferred_element_type=jnp.float32)
                + b_ref[...]
            )
            o_ref[...] = g_ref[...] * proj
        extra_in = (t_prev,)
        extra_specs = (pl.BlockSpec(memory_space=pl.ANY),)
        aliases = {4: 0}

    return pl.pallas_call(
        body,
        grid=(Es // block_e,),
        in_specs=[
            pl.BlockSpec((block_e, D), lambda i: (i, 0)),
            pl.BlockSpec((block_e, K), lambda i: (i, 0)),
            pl.BlockSpec((K, D), lambda i: (0, 0)),
            pl.BlockSpec((1, D), lambda i: (0, 0)),
            *extra_specs,
        ],
        out_specs=pl.BlockSpec((block_e, D), lambda i: (i + off_blocks, 0)),
        out_shape=jax.ShapeDtypeStruct((E, D), jnp.float32),
        input_output_aliases=aliases,
    )(g_slab, ea_slab, W, b2d, *extra_in)


_N_SLABS = 2
_BLOCK_E = 8000


def kernel(senders, receivers, edge_attr, x, W, b):
    E = senders.shape[0]
    N, D = x.shape
    senders = senders.astype(jnp.int32)
    receivers = receivers.astype(jnp.int32)
    b2d = b.reshape(1, D)

    Es = E // _N_SLABS
    sc = _gather_sum_call(Es, N, D)
    gs = [
        sc(x, senders[s * Es:(s + 1) * Es], receivers[s * Es:(s + 1) * Es])
        for s in range(_N_SLABS)
    ]
    blocks_per_slab = Es // _BLOCK_E
    t = None
    for s in range(_N_SLABS):
        t = _combine_slab(
            gs[s], edge_attr[s * Es:(s + 1) * Es], W, b2d, t,
            E, s * blocks_per_slab, _BLOCK_E,
        )
    return t
